# baseline (speedup 1.0000x reference)
import functools

import jax
import jax.numpy as jnp
from jax import lax
from jax.experimental import pallas as pl
from jax.experimental.pallas import tpu as pltpu
from jax.experimental.pallas import tpu_sc as plsc

B = 16384
IN = 512
OUT = 1024
K = 16
BLK = 256
NSC = 7168
NW = 32
RPW = NSC // NW
NCHUNK = OUT // 16

_NEG_INF = float("-inf")


def _mlp3(x, W1, b1, W2, b2, W3, b3):
    h = jnp.maximum(jnp.dot(x, W1, preferred_element_type=jnp.float32) + b1, 0.0)
    h = jnp.maximum(jnp.dot(h, W2, preferred_element_type=jnp.float32) + b2, 0.0)
    return jnp.dot(h, W3, preferred_element_type=jnp.float32) + b3


def _topk_gather(scores, est, iota_f):
    s = scores
    cols = []
    for _ in range(K):
        m = jnp.max(s, axis=1, keepdims=True)
        cand = jnp.where(s == m, iota_f, 2048.0)
        j = jnp.min(cand, axis=1, keepdims=True)
        sel = cand == j
        cols.append(jnp.sum(jnp.where(sel, est, 0.0), axis=1, keepdims=True))
        s = jnp.where(sel, _NEG_INF, s)
    return jnp.concatenate(cols, axis=1)


def _weighter(x, e, wW1x, wW1e, wb1, wW2, wb2, wW3, wb3):
    h = jnp.dot(x, wW1x, preferred_element_type=jnp.float32)
    h = h + jnp.dot(e, wW1e, preferred_element_type=jnp.float32) + wb1
    h = jnp.maximum(h, 0.0)
    h = jnp.maximum(jnp.dot(h, wW2, preferred_element_type=jnp.float32) + wb2, 0.0)
    logits = jnp.dot(h, wW3, preferred_element_type=jnp.float32) + wb3
    logits = logits - jnp.max(logits, axis=1, keepdims=True)
    p = jnp.exp(logits)
    w = p / jnp.sum(p, axis=1, keepdims=True)
    return jnp.sum(e * w, axis=1, keepdims=True)


def _ranker_body(x_ref,
                 roW1, rob1, roW2, rob2, roW3, rob3,
                 ruW1, rub1, ruW2, rub2, ruW3, rub3,
                 so_ref, su_ref):
    x = x_ref[...]
    so_ref[...] = _mlp3(x, roW1[...], rob1[...], roW2[...], rob2[...],
                        roW3[...], rob3[...])
    su_ref[...] = _mlp3(x, ruW1[...], rub1[...], ruW2[...], rub2[...],
                        ruW3[...], rub3[...])


def _tail_tc_body(x_ref, est_ref,
                  roW1, rob1, roW2, rob2, roW3, rob3,
                  ruW1, rub1, ruW2, rub2, ruW3, rub3,
                  wW1x, wW1e, wb1, wW2, wb2, wW3, wb3,
                  logd_ref):
    x = x_ref[...]
    est = est_ref[...]
    so = _mlp3(x, roW1[...], rob1[...], roW2[...], rob2[...],
               roW3[...], rob3[...])
    su = _mlp3(x, ruW1[...], rub1[...], ruW2[...], rub2[...],
               ruW3[...], rub3[...])
    iota_f = jax.lax.broadcasted_iota(jnp.int32, (BLK, OUT), 1).astype(jnp.float32)
    e_over = _topk_gather(so, est, iota_f)
    e_under = _topk_gather(su, est, iota_f)
    e = jnp.concatenate([e_over, e_under], axis=1)
    logd_ref[...] = _weighter(x, e, wW1x[...], wW1e[...], wb1[...],
                              wW2[...], wb2[...], wW3[...], wb3[...])


def _tail_sc_body(x_ref, e_ref,
                  wW1x, wW1e, wb1, wW2, wb2, wW3, wb3,
                  logd_ref):
    logd_ref[...] = _weighter(x_ref[...], e_ref[...], wW1x[...], wW1e[...],
                              wb1[...], wW2[...], wb2[...], wW3[...], wb3[...])


RB = 8
NB = RPW // RB


_GDN = lax.GatherDimensionNumbers(
    offset_dims=(), collapsed_slice_dims=(0,), start_index_map=(0,))


def _permute(x, idxv):
    return lax.gather(x, idxv[:, None], _GDN, (1,),
                      mode=lax.GatherScatterMode.PROMISE_IN_BOUNDS)


def _merge16(a, b, lane):
    av, ai = a
    bv, bi = b
    rv = lax.rev(bv, (0,))
    ri = lax.rev(bi, (0,))
    ge = av >= rv
    hv = jnp.where(ge, av, rv)
    hi = jnp.where(ge, ai, ri)
    for d in (8, 4, 2, 1):
        pidx = lane ^ d
        pv = _permute(hv, pidx)
        pi = _permute(hi, pidx)
        want_max = (lane & d) != 0
        choose_h = (hv >= pv) == want_max
        hv = jnp.where(choose_h, hv, pv)
        hi = jnp.where(choose_h, hi, pi)
    return hv, hi


def _tree_topk(load_chunk, base_iota):
    def leaf(c):
        idx = base_iota + (c * 16)
        return lax.sort((load_chunk(c), idx), dimension=0, num_keys=1)

    def reduce_range(lo, hi):
        if hi - lo == 1:
            return leaf(lo)
        mid = (lo + hi) // 2
        return _merge16(reduce_range(lo, mid), reduce_range(mid, hi), base_iota)

    return reduce_range(0, NCHUNK)


def _sc_body(so_hbm, su_hbm, est_hbm, out_hbm, so_v, su_v, est_v, out_v,
             sem0, sem1):
    wid = lax.axis_index("s") * 2 + lax.axis_index("c")
    base = wid * RPW
    base_iota = lax.iota(jnp.int32, 16)
    sems = (sem0, sem1)

    def copies(i, b):
        r0 = base + i * RB
        return [
            pltpu.make_async_copy(so_hbm.at[pl.ds(r0, RB)], so_v.at[b], sems[b]),
            pltpu.make_async_copy(su_hbm.at[pl.ds(r0, RB)], su_v.at[b], sems[b]),
            pltpu.make_async_copy(est_hbm.at[pl.ds(r0, RB)], est_v.at[b], sems[b]),
        ]

    def issue(i, b):
        for cp in copies(i, b):
            cp.start()

    def wait(i, b):
        for cp in copies(i, b):
            cp.wait()

    issue(0, 0)
    issue(1, 1)

    def process_batch(i, b):
        def row_step(rr, carry):
            _, tio = _tree_topk(lambda c: so_v[b, rr, pl.ds(c * 16, 16)], base_iota)
            _, tiu = _tree_topk(lambda c: su_v[b, rr, pl.ds(c * 16, 16)], base_iota)
            io = lax.rev(tio, (0,))
            iu = lax.rev(tiu, (0,))
            rsplat = jnp.full((16,), rr, jnp.int32)
            bsplat = jnp.full((16,), b, jnp.int32)
            eo = plsc.load_gather(est_v, [bsplat, rsplat, io])
            eu = plsc.load_gather(est_v, [bsplat, rsplat, iu])
            orow = i * RB + rr
            out_v[orow, pl.ds(0, 16)] = eo
            out_v[orow, pl.ds(16, 16)] = eu
            return carry

        lax.fori_loop(0, RB, row_step, 0)

    def outer(g, carry):
        for b in range(2):
            i = g * 2 + b
            wait(i, b)
            process_batch(i, b)

            @pl.when(i + 2 < NB)
            def _():
                issue(i + 2, b)

        return carry

    lax.fori_loop(0, NB // 2, outer, 0)
    pltpu.sync_copy(out_v, out_hbm.at[pl.ds(base, RPW)])


def _make_sc_call():
    mesh = plsc.VectorSubcoreMesh(core_axis_name="c", subcore_axis_name="s")
    return pl.kernel(
        _sc_body,
        out_type=jax.ShapeDtypeStruct((NSC, 2 * K), jnp.float32),
        mesh=mesh,
        compiler_params=pltpu.CompilerParams(needs_layout_passes=False),
        scratch_types=[
            pltpu.VMEM((2, RB, OUT), jnp.float32),
            pltpu.VMEM((2, RB, OUT), jnp.float32),
            pltpu.VMEM((2, RB, OUT), jnp.float32),
            pltpu.VMEM((RPW, 2 * K), jnp.float32),
            pltpu.SemaphoreType.DMA,
            pltpu.SemaphoreType.DMA,
        ],
    )


_sc_call = _make_sc_call()


@jax.jit
def _run(x, estimated_logd, *params):
    (roW1, rob1, roW2, rob2, roW3, rob3,
     ruW1, rub1, ruW2, rub2, ruW3, rub3,
     wW1x, wW1e, wb1, wW2, wb2, wW3, wb3) = params
    rk = (roW1, rob1, roW2, rob2, roW3, rob3,
          ruW1, rub1, ruW2, rub2, ruW3, rub3)
    wt = (wW1x, wW1e, wb1, wW2, wb2, wW3, wb3)

    full = lambda a: pl.BlockSpec(a.shape, lambda i: (0,) * a.ndim)
    row = lambda nc: pl.BlockSpec((BLK, nc), lambda i: (i, 0))
    nsc_blk = NSC // BLK

    so, su = pl.pallas_call(
        _ranker_body,
        grid=(B // BLK,),
        in_specs=[row(IN)] + [full(a) for a in rk],
        out_specs=[row(OUT), row(OUT)],
        out_shape=[jax.ShapeDtypeStruct((B, OUT), jnp.float32)] * 2,
    )(x, *rk)

    e_sc = _sc_call(so, su, estimated_logd)

    off = lambda nc: pl.BlockSpec((BLK, nc), lambda i: (i + nsc_blk, 0))
    logd_tc = pl.pallas_call(
        _tail_tc_body,
        grid=((B - NSC) // BLK,),
        in_specs=[off(IN), off(OUT)] + [full(a) for a in rk + wt],
        out_specs=pl.BlockSpec((BLK, 1), lambda i: (i, 0)),
        out_shape=jax.ShapeDtypeStruct((B - NSC, 1), jnp.float32),
    )(x, estimated_logd, *rk, *wt)

    logd_sc = pl.pallas_call(
        _tail_sc_body,
        grid=(nsc_blk,),
        in_specs=[row(IN), row(2 * K)] + [full(a) for a in wt],
        out_specs=pl.BlockSpec((BLK, 1), lambda i: (i, 0)),
        out_shape=jax.ShapeDtypeStruct((NSC, 1), jnp.float32),
    )(x, e_sc, *wt)

    logd = jnp.concatenate([logd_sc, logd_tc], axis=0)
    return so, su, logd


def kernel(x, estimated_logd, ro_W1, ro_b1, ro_W2, ro_b2, ro_W3, ro_b3,
           ru_W1, ru_b1, ru_W2, ru_b2, ru_W3, ru_b3,
           w_W1, w_b1, w_W2, w_b2, w_W3, w_b3):
    r2 = lambda b: b.reshape(1, -1)
    so, su, logd = _run(
        x, estimated_logd,
        ro_W1, r2(ro_b1), ro_W2, r2(ro_b2), ro_W3, r2(ro_b3),
        ru_W1, r2(ru_b1), ru_W2, r2(ru_b2), ru_W3, r2(ru_b3),
        w_W1[:IN], w_W1[IN:], r2(w_b1), w_W2, r2(w_b2), w_W3, r2(w_b3),
    )
    return (so, su, logd.reshape(B))


# speedup vs baseline: 2.1410x; 2.1410x over previous
import functools

import jax
import jax.numpy as jnp
from jax import lax
from jax.experimental import pallas as pl
from jax.experimental.pallas import tpu as pltpu
from jax.experimental.pallas import tpu_sc as plsc

B = 16384
IN = 512
OUT = 1024
K = 16
BLK = 256
NSC = 7168
NW = 32
RPW = NSC // NW
NCHUNK = OUT // 16

_NEG_INF = float("-inf")


def _mlp3(x, W1, b1, W2, b2, W3, b3):
    h = jnp.maximum(jnp.dot(x, W1, preferred_element_type=jnp.float32) + b1, 0.0)
    h = jnp.maximum(jnp.dot(h, W2, preferred_element_type=jnp.float32) + b2, 0.0)
    return jnp.dot(h, W3, preferred_element_type=jnp.float32) + b3


def _topk_gather(scores, est, iota_f):
    s = scores
    cols = []
    for _ in range(K):
        m = jnp.max(s, axis=1, keepdims=True)
        cand = jnp.where(s == m, iota_f, 2048.0)
        j = jnp.min(cand, axis=1, keepdims=True)
        sel = cand == j
        cols.append(jnp.sum(jnp.where(sel, est, 0.0), axis=1, keepdims=True))
        s = jnp.where(sel, _NEG_INF, s)
    return jnp.concatenate(cols, axis=1)


def _weighter(x, e, wW1x, wW1e, wb1, wW2, wb2, wW3, wb3):
    h = jnp.dot(x, wW1x, preferred_element_type=jnp.float32)
    h = h + jnp.dot(e, wW1e, preferred_element_type=jnp.float32) + wb1
    h = jnp.maximum(h, 0.0)
    h = jnp.maximum(jnp.dot(h, wW2, preferred_element_type=jnp.float32) + wb2, 0.0)
    logits = jnp.dot(h, wW3, preferred_element_type=jnp.float32) + wb3
    logits = logits - jnp.max(logits, axis=1, keepdims=True)
    p = jnp.exp(logits)
    w = p / jnp.sum(p, axis=1, keepdims=True)
    return jnp.sum(e * w, axis=1, keepdims=True)


def _ranker_body(x_ref,
                 roW1, rob1, roW2, rob2, roW3, rob3,
                 ruW1, rub1, ruW2, rub2, ruW3, rub3,
                 so_ref, su_ref):
    x = x_ref[...]
    so_ref[...] = _mlp3(x, roW1[...], rob1[...], roW2[...], rob2[...],
                        roW3[...], rob3[...])
    su_ref[...] = _mlp3(x, ruW1[...], rub1[...], ruW2[...], rub2[...],
                        ruW3[...], rub3[...])


def _tail_tc_body(x_ref, est_ref,
                  roW1, rob1, roW2, rob2, roW3, rob3,
                  ruW1, rub1, ruW2, rub2, ruW3, rub3,
                  wW1x, wW1e, wb1, wW2, wb2, wW3, wb3,
                  logd_ref):
    x = x_ref[...]
    est = est_ref[...]
    so = _mlp3(x, roW1[...], rob1[...], roW2[...], rob2[...],
               roW3[...], rob3[...])
    su = _mlp3(x, ruW1[...], rub1[...], ruW2[...], rub2[...],
               ruW3[...], rub3[...])
    iota_f = jax.lax.broadcasted_iota(jnp.int32, (BLK, OUT), 1).astype(jnp.float32)
    e_over = _topk_gather(so, est, iota_f)
    e_under = _topk_gather(su, est, iota_f)
    e = jnp.concatenate([e_over, e_under], axis=1)
    logd_ref[...] = _weighter(x, e, wW1x[...], wW1e[...], wb1[...],
                              wW2[...], wb2[...], wW3[...], wb3[...])


def _tail_sc_body(x_ref, e_ref,
                  wW1x, wW1e, wb1, wW2, wb2, wW3, wb3,
                  logd_ref):
    logd_ref[...] = _weighter(x_ref[...], e_ref[...], wW1x[...], wW1e[...],
                              wb1[...], wW2[...], wb2[...], wW3[...], wb3[...])


RB = 8
NB = RPW // RB


def _merge16(a, b):
    av, ai = a
    bv, bi = b
    rv = lax.rev(bv, (0,))
    ri = lax.rev(bi, (0,))
    ge = av >= rv
    hv = jnp.where(ge, av, rv)
    hi = jnp.where(ge, ai, ri)
    return lax.sort((hv, hi), dimension=0, num_keys=1)


def _tree_topk(load_chunk, base_iota):
    def leaf(c):
        idx = base_iota + (c * 16)
        return lax.sort((load_chunk(c), idx), dimension=0, num_keys=1)

    def reduce_range(lo, hi):
        if hi - lo == 1:
            return leaf(lo)
        mid = (lo + hi) // 2
        return _merge16(reduce_range(lo, mid), reduce_range(mid, hi))

    return reduce_range(0, NCHUNK)


def _sc_body(so_hbm, su_hbm, est_hbm, out_hbm, so_v, su_v, est_v, out_v,
             sem0, sem1):
    wid = lax.axis_index("s") * 2 + lax.axis_index("c")
    base = wid * RPW
    base_iota = lax.iota(jnp.int32, 16)
    sems = (sem0, sem1)

    def copies(i, b):
        r0 = base + i * RB
        return [
            pltpu.make_async_copy(so_hbm.at[pl.ds(r0, RB)], so_v.at[b], sems[b]),
            pltpu.make_async_copy(su_hbm.at[pl.ds(r0, RB)], su_v.at[b], sems[b]),
            pltpu.make_async_copy(est_hbm.at[pl.ds(r0, RB)], est_v.at[b], sems[b]),
        ]

    def issue(i, b):
        for cp in copies(i, b):
            cp.start()

    def wait(i, b):
        for cp in copies(i, b):
            cp.wait()

    issue(0, 0)
    issue(1, 1)

    def process_batch(i, b):
        def row_step(rr, carry):
            _, tio = _tree_topk(lambda c: so_v[b, rr, pl.ds(c * 16, 16)], base_iota)
            _, tiu = _tree_topk(lambda c: su_v[b, rr, pl.ds(c * 16, 16)], base_iota)
            io = lax.rev(tio, (0,))
            iu = lax.rev(tiu, (0,))
            rsplat = jnp.full((16,), rr, jnp.int32)
            bsplat = jnp.full((16,), b, jnp.int32)
            eo = plsc.load_gather(est_v, [bsplat, rsplat, io])
            eu = plsc.load_gather(est_v, [bsplat, rsplat, iu])
            orow = i * RB + rr
            out_v[orow, pl.ds(0, 16)] = eo
            out_v[orow, pl.ds(16, 16)] = eu
            return carry

        lax.fori_loop(0, RB, row_step, 0)

    def outer(g, carry):
        for b in range(2):
            i = g * 2 + b
            wait(i, b)
            process_batch(i, b)

            @pl.when(i + 2 < NB)
            def _():
                issue(i + 2, b)

        return carry

    lax.fori_loop(0, NB // 2, outer, 0)
    pltpu.sync_copy(out_v, out_hbm.at[pl.ds(base, RPW)])


def _make_sc_call():
    mesh = plsc.VectorSubcoreMesh(core_axis_name="c", subcore_axis_name="s")
    return pl.kernel(
        _sc_body,
        out_type=jax.ShapeDtypeStruct((NSC, 2 * K), jnp.float32),
        mesh=mesh,
        compiler_params=pltpu.CompilerParams(needs_layout_passes=False),
        scratch_types=[
            pltpu.VMEM((2, RB, OUT), jnp.float32),
            pltpu.VMEM((2, RB, OUT), jnp.float32),
            pltpu.VMEM((2, RB, OUT), jnp.float32),
            pltpu.VMEM((RPW, 2 * K), jnp.float32),
            pltpu.SemaphoreType.DMA,
            pltpu.SemaphoreType.DMA,
        ],
    )


_sc_call = _make_sc_call()


@jax.jit
def _run(x, estimated_logd, *params):
    (roW1, rob1, roW2, rob2, roW3, rob3,
     ruW1, rub1, ruW2, rub2, ruW3, rub3,
     wW1x, wW1e, wb1, wW2, wb2, wW3, wb3) = params
    rk = (roW1, rob1, roW2, rob2, roW3, rob3,
          ruW1, rub1, ruW2, rub2, ruW3, rub3)
    wt = (wW1x, wW1e, wb1, wW2, wb2, wW3, wb3)

    full = lambda a: pl.BlockSpec(a.shape, lambda i: (0,) * a.ndim)
    row = lambda nc: pl.BlockSpec((BLK, nc), lambda i: (i, 0))
    nsc_blk = NSC // BLK

    so, su = pl.pallas_call(
        _ranker_body,
        grid=(B // BLK,),
        in_specs=[row(IN)] + [full(a) for a in rk],
        out_specs=[row(OUT), row(OUT)],
        out_shape=[jax.ShapeDtypeStruct((B, OUT), jnp.float32)] * 2,
    )(x, *rk)

    e_sc = _sc_call(so, su, estimated_logd)

    off = lambda nc: pl.BlockSpec((BLK, nc), lambda i: (i + nsc_blk, 0))
    logd_tc = pl.pallas_call(
        _tail_tc_body,
        grid=((B - NSC) // BLK,),
        in_specs=[off(IN), off(OUT)] + [full(a) for a in rk + wt],
        out_specs=pl.BlockSpec((BLK, 1), lambda i: (i, 0)),
        out_shape=jax.ShapeDtypeStruct((B - NSC, 1), jnp.float32),
    )(x, estimated_logd, *rk, *wt)

    logd_sc = pl.pallas_call(
        _tail_sc_body,
        grid=(nsc_blk,),
        in_specs=[row(IN), row(2 * K)] + [full(a) for a in wt],
        out_specs=pl.BlockSpec((BLK, 1), lambda i: (i, 0)),
        out_shape=jax.ShapeDtypeStruct((NSC, 1), jnp.float32),
    )(x, e_sc, *wt)

    logd = jnp.concatenate([logd_sc, logd_tc], axis=0)
    return so, su, logd


def kernel(x, estimated_logd, ro_W1, ro_b1, ro_W2, ro_b2, ro_W3, ro_b3,
           ru_W1, ru_b1, ru_W2, ru_b2, ru_W3, ru_b3,
           w_W1, w_b1, w_W2, w_b2, w_W3, w_b3):
    r2 = lambda b: b.reshape(1, -1)
    so, su, logd = _run(
        x, estimated_logd,
        ro_W1, r2(ro_b1), ro_W2, r2(ro_b2), ro_W3, r2(ro_b3),
        ru_W1, r2(ru_b1), ru_W2, r2(ru_b2), ru_W3, r2(ru_b3),
        w_W1[:IN], w_W1[IN:], r2(w_b1), w_W2, r2(w_b2), w_W3, r2(w_b3),
    )
    return (so, su, logd.reshape(B))
